# Initial kernel scaffold; baseline (speedup 1.0000x reference)
import jax
import jax.numpy as jnp
from jax.experimental import pallas as pl


def kernel(X, E, gw1, gb1, gw2, gb2, ew1, eb1, ew2, eb2, iw1, ib1, iw2, ib2, coef_g, coef_w, coef_e):
    raise NotImplementedError("write your pallas kernel here")


# speedup vs baseline: 2.0944x; 2.0944x over previous
import jax
import jax.numpy as jnp
from jax.experimental import pallas as pl
from jax.experimental.pallas import tpu as pltpu

_BN_EPS = 1e-5


def _mlp_residual(slab, w1, b1, w2, b2, h):
  acc = jnp.broadcast_to(b2, slab.shape)
  for kk in range(h):
    z = slab * w1[kk:kk + 1, :] + b1[kk:kk + 1, :]
    acc = acc + w2[kk:kk + 1, :] * jnp.maximum(z, 0.0)
  return slab + acc


def _make_pass1(r, q, h, W, tnf, Tc, n_true):
  P = r * (q + 1) + q
  need_mask = (2 * Tc * tnf) != n_true

  def body(x_ref, e_ref, w1_ref, b1_ref, w2_ref, b2_ref, slab_ref, stats_ref):
    c = pl.program_id(0)
    i = pl.program_id(1)
    X = x_ref[...]
    E = e_ref[...]
    pieces = [X]
    for e in range(q):
      pieces.append(E[:, e:e + 1] * X)
    pieces.append(E)
    if W > P:
      pieces.append(jnp.zeros((tnf, W - P), jnp.float32))
    slab = jnp.concatenate(pieces, axis=1)
    pre = _mlp_residual(slab, w1_ref[...], b1_ref[...], w2_ref[...],
                        b2_ref[...], h)
    slab_ref[...] = pre

    if need_mask:
      row = (c * Tc + i) * tnf + jax.lax.broadcasted_iota(
          jnp.int32, (tnf, W), 0)
      pre = jnp.where(row < n_true, pre, 0.0)
    s = jnp.sum(pre, axis=0, keepdims=True)
    ss = jnp.sum(pre * pre, axis=0, keepdims=True)
    new = jnp.concatenate([s, ss], axis=0).reshape(1, 2, W)

    @pl.when(i == 0)
    def _():
      stats_ref[...] = new

    @pl.when(i > 0)
    def _():
      stats_ref[...] = stats_ref[...] + new

  return body


def _make_pass2(r, q, W, out_w, inv_n):
  def body(slab_ref, stats_ref, coef_ref, out_ref):
    st = stats_ref[...]
    s = st[0, 0:1, :] + st[1, 0:1, :]
    ss = st[0, 1:2, :] + st[1, 1:2, :]
    mean = s * inv_n
    var = jnp.maximum(ss * inv_n - mean * mean, 0.0)
    a = jax.lax.rsqrt(var + _BN_EPS) * coef_ref[...]
    b = mean * a
    res = slab_ref[...] * a - b
    pred = jnp.sum(res, axis=1, keepdims=True)
    psep = res[:, 0:r]
    for e in range(q):
      psep = psep + res[:, (e + 1) * r:(e + 2) * r]
    t = res.shape[0]
    pad = out_w - 1 - r
    pieces = [pred, psep]
    if pad:
      pieces.append(jnp.zeros((t, pad), jnp.float32))
    out_ref[...] = jnp.concatenate(pieces, axis=1)

  return body


def _pack_weights(r, q, h, W, gw1, gb1, gw2, gb2, ew1, eb1, ew2, eb2,
                  iw1, ib1, iw2, ib2, coef_g, coef_w, coef_e):
  P = r * (q + 1) + q

  def padh(a, ha):
    if ha == h:
      return a
    return jnp.concatenate([a, jnp.zeros((h - ha, a.shape[1]), a.dtype)], 0)

  def lanes(g_part, i_part, e_part):
    x = jnp.concatenate([g_part, i_part, e_part], axis=1)
    if W > P:
      x = jnp.concatenate([x, jnp.zeros((x.shape[0], W - P), x.dtype)], 1)
    return x.astype(jnp.float32)

  h_g, h_e, h_i = gw1.shape[0], ew1.shape[0], iw1.shape[1]
  iw1f = jnp.transpose(iw1, (1, 0, 2)).reshape(h_i, q * r)
  ib1f = jnp.transpose(ib1, (1, 0, 2)).reshape(h_i, q * r)
  iw2f = jnp.transpose(iw2, (1, 0, 2)).reshape(h_i, q * r)
  ib2f = jnp.transpose(ib2, (1, 0, 2)).reshape(1, q * r)
  w1 = lanes(padh(gw1, h_g), padh(iw1f, h_i), padh(ew1, h_e))
  b1 = lanes(padh(gb1, h_g), padh(ib1f, h_i), padh(eb1, h_e))
  w2 = lanes(padh(gw2, h_g), padh(iw2f, h_i), padh(ew2, h_e))
  b2 = lanes(gb2, ib2f, eb2)
  coef = lanes(coef_g, coef_w.reshape(1, q * r), coef_e)
  return w1, b1, w2, b2, coef


def kernel(X, E, gw1, gb1, gw2, gb2, ew1, eb1, ew2, eb2,
           iw1, ib1, iw2, ib2, coef_g, coef_w, coef_e):
  X = jnp.asarray(X, jnp.float32)
  E = jnp.asarray(E, jnp.float32)
  n, r = X.shape
  q = E.shape[1]
  h = max(gw1.shape[0], ew1.shape[0], iw1.shape[1])
  P = r * (q + 1) + q
  W = ((P + 127) // 128) * 128
  out_w = ((1 + r + 127) // 128) * 128

  w1, b1, w2, b2, coef = _pack_weights(
      r, q, h, W, gw1, gb1, gw2, gb2, ew1, eb1, ew2, eb2,
      iw1, ib1, iw2, ib2, coef_g, coef_w, coef_e)

  if n >= 4096:
    tnf = 2048
  else:
    tnf = max(8, ((n + 1) // 2 + 7) // 8 * 8)
  n_pad = -(-n // (2 * tnf)) * 2 * tnf
  Tc = n_pad // (2 * tnf)
  if n_pad != n:
    X = jnp.concatenate([X, jnp.zeros((n_pad - n, r), jnp.float32)], axis=0)
    E = jnp.concatenate([E, jnp.zeros((n_pad - n, q), jnp.float32)], axis=0)

  vmem_limit = 56 * 2**20

  slab, stats = pl.pallas_call(
      _make_pass1(r, q, h, W, tnf, Tc, n),
      out_shape=(jax.ShapeDtypeStruct((n_pad, W), jnp.float32),
                 jax.ShapeDtypeStruct((2, 2, W), jnp.float32)),
      grid=(2, Tc),
      in_specs=[
          pl.BlockSpec((tnf, r), lambda c, i: (c * Tc + i, 0)),
          pl.BlockSpec((tnf, q), lambda c, i: (c * Tc + i, 0)),
          pl.BlockSpec((h, W), lambda c, i: (0, 0)),
          pl.BlockSpec((h, W), lambda c, i: (0, 0)),
          pl.BlockSpec((h, W), lambda c, i: (0, 0)),
          pl.BlockSpec((1, W), lambda c, i: (0, 0)),
      ],
      out_specs=(pl.BlockSpec((tnf, W), lambda c, i: (c * Tc + i, 0)),
                 pl.BlockSpec((1, 2, W), lambda c, i: (c, 0, 0))),
      compiler_params=pltpu.CompilerParams(
          dimension_semantics=("parallel", "arbitrary"),
          vmem_limit_bytes=vmem_limit),
  )(X, E, w1, b1, w2, b2)

  out = pl.pallas_call(
      _make_pass2(r, q, W, out_w, 1.0 / float(n)),
      out_shape=jax.ShapeDtypeStruct((n_pad, out_w), jnp.float32),
      grid=(2, Tc),
      in_specs=[
          pl.BlockSpec((tnf, W), lambda c, i: (c * Tc + i, 0)),
          pl.BlockSpec((2, 2, W), lambda c, i: (0, 0, 0)),
          pl.BlockSpec((1, W), lambda c, i: (0, 0)),
      ],
      out_specs=pl.BlockSpec((tnf, out_w), lambda c, i: (c * Tc + i, 0)),
      compiler_params=pltpu.CompilerParams(
          dimension_semantics=("parallel", "arbitrary"),
          vmem_limit_bytes=vmem_limit),
  )(slab, stats, coef)

  return out[:n, 0:1], out[:n, 1:1 + r]


# speedup vs baseline: 3.4200x; 1.6330x over previous
import jax
import jax.numpy as jnp
from jax.experimental import pallas as pl
from jax.experimental.pallas import tpu as pltpu

_BN_EPS = 1e-5


def _mlp_residual(slab, w1, b1, w2, b2, h):
  sb = slab.astype(jnp.bfloat16)
  zero = jnp.bfloat16(0.0)
  acc = jnp.broadcast_to(b2, sb.shape)
  for kk in range(h):
    z = sb * w1[kk:kk + 1, :] + b1[kk:kk + 1, :]
    acc = acc + w2[kk:kk + 1, :] * jnp.maximum(z, zero)
  return slab + acc.astype(jnp.float32)


def _make_pass1(r, q, h, W, tnf, Tc, n_true):
  P = r * (q + 1) + q
  need_mask = (2 * Tc * tnf) != n_true

  def body(x_ref, e_ref, w1_ref, b1_ref, w2_ref, b2_ref, slab_ref, stats_ref):
    c = pl.program_id(0)
    i = pl.program_id(1)
    X = x_ref[...]
    E = e_ref[...]
    pieces = [X]
    for e in range(q):
      pieces.append(E[:, e:e + 1] * X)
    pieces.append(E)
    if W > P:
      pieces.append(jnp.zeros((tnf, W - P), jnp.float32))
    slab = jnp.concatenate(pieces, axis=1)
    pre = _mlp_residual(slab, w1_ref[...], b1_ref[...], w2_ref[...],
                        b2_ref[...], h)
    slab_ref[...] = pre

    if need_mask:
      row = (c * Tc + i) * tnf + jax.lax.broadcasted_iota(
          jnp.int32, (tnf, W), 0)
      pre = jnp.where(row < n_true, pre, 0.0)
    s = jnp.sum(pre, axis=0, keepdims=True)
    ss = jnp.sum(pre * pre, axis=0, keepdims=True)
    new = jnp.concatenate([s, ss], axis=0).reshape(1, 2, W)

    @pl.when(i == 0)
    def _():
      stats_ref[...] = new

    @pl.when(i > 0)
    def _():
      stats_ref[...] = stats_ref[...] + new

  return body


def _make_pass2(r, q, W, out_w, inv_n):
  def body(slab_ref, stats_ref, coef_ref, out_ref):
    st = stats_ref[...]
    s = st[0, 0:1, :] + st[1, 0:1, :]
    ss = st[0, 1:2, :] + st[1, 1:2, :]
    mean = s * inv_n
    var = jnp.maximum(ss * inv_n - mean * mean, 0.0)
    a = jax.lax.rsqrt(var + _BN_EPS) * coef_ref[...]
    b = mean * a
    res = slab_ref[...] * a - b
    pred = jnp.sum(res, axis=1, keepdims=True)
    psep = res[:, 0:r]
    for e in range(q):
      psep = psep + res[:, (e + 1) * r:(e + 2) * r]
    t = res.shape[0]
    pad = out_w - 1 - r
    pieces = [pred, psep]
    if pad:
      pieces.append(jnp.zeros((t, pad), jnp.float32))
    out_ref[...] = jnp.concatenate(pieces, axis=1)

  return body


def _pack_weights(r, q, h, W, gw1, gb1, gw2, gb2, ew1, eb1, ew2, eb2,
                  iw1, ib1, iw2, ib2, coef_g, coef_w, coef_e):
  P = r * (q + 1) + q

  def padh(a, ha):
    if ha == h:
      return a
    return jnp.concatenate([a, jnp.zeros((h - ha, a.shape[1]), a.dtype)], 0)

  def lanes(g_part, i_part, e_part):
    x = jnp.concatenate([g_part, i_part, e_part], axis=1)
    if W > P:
      x = jnp.concatenate([x, jnp.zeros((x.shape[0], W - P), x.dtype)], 1)
    return x.astype(jnp.float32)

  h_g, h_e, h_i = gw1.shape[0], ew1.shape[0], iw1.shape[1]
  iw1f = jnp.transpose(iw1, (1, 0, 2)).reshape(h_i, q * r)
  ib1f = jnp.transpose(ib1, (1, 0, 2)).reshape(h_i, q * r)
  iw2f = jnp.transpose(iw2, (1, 0, 2)).reshape(h_i, q * r)
  ib2f = jnp.transpose(ib2, (1, 0, 2)).reshape(1, q * r)
  w1 = lanes(padh(gw1, h_g), padh(iw1f, h_i), padh(ew1, h_e))
  b1 = lanes(padh(gb1, h_g), padh(ib1f, h_i), padh(eb1, h_e))
  w2 = lanes(padh(gw2, h_g), padh(iw2f, h_i), padh(ew2, h_e))
  b2 = lanes(gb2, ib2f, eb2)
  coef = lanes(coef_g, coef_w.reshape(1, q * r), coef_e)
  return w1, b1, w2, b2, coef


def kernel(X, E, gw1, gb1, gw2, gb2, ew1, eb1, ew2, eb2,
           iw1, ib1, iw2, ib2, coef_g, coef_w, coef_e):
  X = jnp.asarray(X, jnp.float32)
  E = jnp.asarray(E, jnp.float32)
  n, r = X.shape
  q = E.shape[1]
  h = max(gw1.shape[0], ew1.shape[0], iw1.shape[1])
  P = r * (q + 1) + q
  W = ((P + 127) // 128) * 128
  out_w = ((1 + r + 127) // 128) * 128

  w1, b1, w2, b2, coef = _pack_weights(
      r, q, h, W, gw1, gb1, gw2, gb2, ew1, eb1, ew2, eb2,
      iw1, ib1, iw2, ib2, coef_g, coef_w, coef_e)
  w1 = w1.astype(jnp.bfloat16)
  b1 = b1.astype(jnp.bfloat16)
  w2 = w2.astype(jnp.bfloat16)
  b2 = b2.astype(jnp.bfloat16)

  if n >= 4096:
    tnf = 2048
  else:
    tnf = max(8, ((n + 1) // 2 + 7) // 8 * 8)
  n_pad = -(-n // (2 * tnf)) * 2 * tnf
  Tc = n_pad // (2 * tnf)
  if n_pad != n:
    X = jnp.concatenate([X, jnp.zeros((n_pad - n, r), jnp.float32)], axis=0)
    E = jnp.concatenate([E, jnp.zeros((n_pad - n, q), jnp.float32)], axis=0)

  vmem_limit = 56 * 2**20

  slab, stats = pl.pallas_call(
      _make_pass1(r, q, h, W, tnf, Tc, n),
      out_shape=(jax.ShapeDtypeStruct((n_pad, W), jnp.float32),
                 jax.ShapeDtypeStruct((2, 2, W), jnp.float32)),
      grid=(2, Tc),
      in_specs=[
          pl.BlockSpec((tnf, r), lambda c, i: (c * Tc + i, 0)),
          pl.BlockSpec((tnf, q), lambda c, i: (c * Tc + i, 0)),
          pl.BlockSpec((h, W), lambda c, i: (0, 0)),
          pl.BlockSpec((h, W), lambda c, i: (0, 0)),
          pl.BlockSpec((h, W), lambda c, i: (0, 0)),
          pl.BlockSpec((1, W), lambda c, i: (0, 0)),
      ],
      out_specs=(pl.BlockSpec((tnf, W), lambda c, i: (c * Tc + i, 0)),
                 pl.BlockSpec((1, 2, W), lambda c, i: (c, 0, 0))),
      compiler_params=pltpu.CompilerParams(
          dimension_semantics=("parallel", "arbitrary"),
          vmem_limit_bytes=vmem_limit),
  )(X, E, w1, b1, w2, b2)

  out = pl.pallas_call(
      _make_pass2(r, q, W, out_w, 1.0 / float(n)),
      out_shape=jax.ShapeDtypeStruct((n_pad, out_w), jnp.float32),
      grid=(2, Tc),
      in_specs=[
          pl.BlockSpec((tnf, W), lambda c, i: (c * Tc + i, 0)),
          pl.BlockSpec((2, 2, W), lambda c, i: (0, 0, 0)),
          pl.BlockSpec((1, W), lambda c, i: (0, 0)),
      ],
      out_specs=pl.BlockSpec((tnf, out_w), lambda c, i: (c * Tc + i, 0)),
      compiler_params=pltpu.CompilerParams(
          dimension_semantics=("parallel", "arbitrary"),
          vmem_limit_bytes=vmem_limit),
  )(slab, stats, coef)

  return out[:n, 0:1], out[:n, 1:1 + r]


# speedup vs baseline: 3.8572x; 1.1278x over previous
import jax
import jax.numpy as jnp
from jax.experimental import pallas as pl
from jax.experimental.pallas import tpu as pltpu

_BN_EPS = 1e-5


def _rup(x, m):
  return ((x + m - 1) // m) * m


def _mlp_residual(slab, w1, b1, w2, b2, h):
  sb = slab.astype(jnp.bfloat16)
  zero = jnp.bfloat16(0.0)
  acc = jnp.broadcast_to(b2, sb.shape)
  for kk in range(h):
    z = sb * w1[kk:kk + 1, :] + b1[kk:kk + 1, :]
    acc = acc + w2[kk:kk + 1, :] * jnp.maximum(z, zero)
  return slab + acc.astype(jnp.float32)


def _make_pass1(r, q, h, g0w, giw, tnf, Tc, n_true):
  W = g0w + q * giw
  need_mask = (2 * Tc * tnf) != n_true

  def body(x_ref, e_ref, w1_ref, b1_ref, w2_ref, b2_ref, slab_ref, stats_ref):
    c = pl.program_id(0)
    i = pl.program_id(1)
    X = x_ref[...]
    E = e_ref[...]
    pieces = [X, E]
    if g0w > r + q:
      pieces.append(jnp.zeros((tnf, g0w - r - q), jnp.float32))
    for e in range(q):
      pieces.append(E[:, e:e + 1] * X)
      if giw > r:
        pieces.append(jnp.zeros((tnf, giw - r), jnp.float32))
    slab = jnp.concatenate(pieces, axis=1)
    pre = _mlp_residual(slab, w1_ref[...], b1_ref[...], w2_ref[...],
                        b2_ref[...], h)
    slab_ref[...] = pre

    if need_mask:
      row = (c * Tc + i) * tnf + jax.lax.broadcasted_iota(
          jnp.int32, (tnf, W), 0)
      pre = jnp.where(row < n_true, pre, 0.0)
    s = jnp.sum(pre, axis=0, keepdims=True)
    ss = jnp.sum(pre * pre, axis=0, keepdims=True)
    new = jnp.concatenate([s, ss], axis=0).reshape(1, 2, W)

    @pl.when(i == 0)
    def _():
      stats_ref[...] = new

    @pl.when(i > 0)
    def _():
      stats_ref[...] = stats_ref[...] + new

  return body


def _make_pass2(r, q, g0w, giw, inv_n):
  def body(slab_ref, stats_ref, coef_ref, pred_ref, psep_ref):
    st = stats_ref[...]
    s = st[0, 0:1, :] + st[1, 0:1, :]
    ss = st[0, 1:2, :] + st[1, 1:2, :]
    mean = s * inv_n
    var = jnp.maximum(ss * inv_n - mean * mean, 0.0)
    a = jax.lax.rsqrt(var + _BN_EPS) * coef_ref[...]
    b = mean * a
    res = slab_ref[...] * a - b
    pred_ref[...] = jnp.sum(res, axis=1, keepdims=True)
    psep = res[:, 0:r]
    for e in range(q):
      off = g0w + e * giw
      psep = psep + res[:, off:off + r]
    psep_ref[...] = psep

  return body


def _pack_weights(r, q, h, g0w, giw, gw1, gb1, gw2, gb2, ew1, eb1, ew2, eb2,
                  iw1, ib1, iw2, ib2, coef_g, coef_w, coef_e):
  def padh(a, ha):
    if ha == h:
      return a
    return jnp.concatenate([a, jnp.zeros((h - ha, a.shape[1]), a.dtype)], 0)

  def lanes(g_part, i_part, e_part):
    rows = g_part.shape[0]
    pieces = [g_part, e_part]
    if g0w > r + q:
      pieces.append(jnp.zeros((rows, g0w - r - q), g_part.dtype))
    for e in range(q):
      pieces.append(i_part[:, e * r:(e + 1) * r])
      if giw > r:
        pieces.append(jnp.zeros((rows, giw - r), g_part.dtype))
    return jnp.concatenate(pieces, axis=1).astype(jnp.float32)

  h_g, h_e, h_i = gw1.shape[0], ew1.shape[0], iw1.shape[1]
  iw1f = jnp.transpose(iw1, (1, 0, 2)).reshape(h_i, q * r)
  ib1f = jnp.transpose(ib1, (1, 0, 2)).reshape(h_i, q * r)
  iw2f = jnp.transpose(iw2, (1, 0, 2)).reshape(h_i, q * r)
  ib2f = jnp.transpose(ib2, (1, 0, 2)).reshape(1, q * r)
  w1 = lanes(padh(gw1, h_g), padh(iw1f, h_i), padh(ew1, h_e))
  b1 = lanes(padh(gb1, h_g), padh(ib1f, h_i), padh(eb1, h_e))
  w2 = lanes(padh(gw2, h_g), padh(iw2f, h_i), padh(ew2, h_e))
  b2 = lanes(gb2, ib2f, eb2)
  coef = lanes(coef_g, coef_w.reshape(1, q * r), coef_e)
  return w1, b1, w2, b2, coef


def kernel(X, E, gw1, gb1, gw2, gb2, ew1, eb1, ew2, eb2,
           iw1, ib1, iw2, ib2, coef_g, coef_w, coef_e):
  X = jnp.asarray(X, jnp.float32)
  E = jnp.asarray(E, jnp.float32)
  n, r = X.shape
  q = E.shape[1]
  h = max(gw1.shape[0], ew1.shape[0], iw1.shape[1])
  g0w = _rup(r + q, 128)
  giw = _rup(r, 128)
  W = g0w + q * giw

  w1, b1, w2, b2, coef = _pack_weights(
      r, q, h, g0w, giw, gw1, gb1, gw2, gb2, ew1, eb1, ew2, eb2,
      iw1, ib1, iw2, ib2, coef_g, coef_w, coef_e)
  w1 = w1.astype(jnp.bfloat16)
  b1 = b1.astype(jnp.bfloat16)
  w2 = w2.astype(jnp.bfloat16)
  b2 = b2.astype(jnp.bfloat16)

  if n >= 4096:
    tnf = 2048
  else:
    tnf = max(8, _rup((n + 1) // 2, 8))
  n_pad = _rup(n, 2 * tnf)
  Tc = n_pad // (2 * tnf)
  if n_pad != n:
    X = jnp.concatenate([X, jnp.zeros((n_pad - n, r), jnp.float32)], axis=0)
    E = jnp.concatenate([E, jnp.zeros((n_pad - n, q), jnp.float32)], axis=0)

  vmem_limit = 56 * 2**20

  slab, stats = pl.pallas_call(
      _make_pass1(r, q, h, g0w, giw, tnf, Tc, n),
      out_shape=(jax.ShapeDtypeStruct((n_pad, W), jnp.float32),
                 jax.ShapeDtypeStruct((2, 2, W), jnp.float32)),
      grid=(2, Tc),
      in_specs=[
          pl.BlockSpec((tnf, r), lambda c, i: (c * Tc + i, 0)),
          pl.BlockSpec((tnf, q), lambda c, i: (c * Tc + i, 0)),
          pl.BlockSpec((h, W), lambda c, i: (0, 0)),
          pl.BlockSpec((h, W), lambda c, i: (0, 0)),
          pl.BlockSpec((h, W), lambda c, i: (0, 0)),
          pl.BlockSpec((1, W), lambda c, i: (0, 0)),
      ],
      out_specs=(pl.BlockSpec((tnf, W), lambda c, i: (c * Tc + i, 0)),
                 pl.BlockSpec((1, 2, W), lambda c, i: (c, 0, 0))),
      compiler_params=pltpu.CompilerParams(
          dimension_semantics=("parallel", "arbitrary"),
          vmem_limit_bytes=vmem_limit),
  )(X, E, w1, b1, w2, b2)

  pred, psep = pl.pallas_call(
      _make_pass2(r, q, g0w, giw, 1.0 / float(n)),
      out_shape=(jax.ShapeDtypeStruct((n_pad, 1), jnp.float32),
                 jax.ShapeDtypeStruct((n_pad, r), jnp.float32)),
      grid=(2, Tc),
      in_specs=[
          pl.BlockSpec((tnf, W), lambda c, i: (c * Tc + i, 0)),
          pl.BlockSpec((2, 2, W), lambda c, i: (0, 0, 0)),
          pl.BlockSpec((1, W), lambda c, i: (0, 0)),
      ],
      out_specs=(pl.BlockSpec((tnf, 1), lambda c, i: (c * Tc + i, 0)),
                 pl.BlockSpec((tnf, r), lambda c, i: (c * Tc + i, 0))),
      compiler_params=pltpu.CompilerParams(
          dimension_semantics=("parallel", "arbitrary"),
          vmem_limit_bytes=vmem_limit),
  )(slab, stats, coef)

  if n_pad != n:
    pred = pred[:n]
    psep = psep[:n]
  return pred, psep


# speedup vs baseline: 3.9499x; 1.0241x over previous
import jax
import jax.numpy as jnp
from jax.experimental import pallas as pl
from jax.experimental.pallas import tpu as pltpu

_BN_EPS = 1e-5


def _rup(x, m):
  return ((x + m - 1) // m) * m


def _mlp_residual(slab, w1, b1, w2, b2, h):
  sb = slab.astype(jnp.bfloat16)
  zero = jnp.bfloat16(0.0)
  acc = jnp.broadcast_to(b2, sb.shape)
  for kk in range(h):
    z = sb * w1[kk:kk + 1, :] + b1[kk:kk + 1, :]
    acc = acc + w2[kk:kk + 1, :] * jnp.maximum(z, zero)
  return slab + acc.astype(jnp.float32)


def _make_pass1(r, q, h, g0w, giw, tnf, Tc, n_true):
  W = g0w + q * giw
  need_mask = (2 * Tc * tnf) != n_true

  def body(x_ref, e_ref, w1_ref, b1_ref, w2_ref, b2_ref, slab_ref, stats_ref):
    c = pl.program_id(0)
    i = pl.program_id(1)
    X = x_ref[...]
    E = e_ref[...]
    pieces = [X, E]
    if g0w > r + q:
      pieces.append(jnp.zeros((tnf, g0w - r - q), jnp.float32))
    for e in range(q):
      pieces.append(E[:, e:e + 1] * X)
      if giw > r:
        pieces.append(jnp.zeros((tnf, giw - r), jnp.float32))
    slab = jnp.concatenate(pieces, axis=1)
    pre = _mlp_residual(slab, w1_ref[...], b1_ref[...], w2_ref[...],
                        b2_ref[...], h)
    slab_ref[...] = pre.astype(jnp.bfloat16)

    if need_mask:
      row = (c * Tc + i) * tnf + jax.lax.broadcasted_iota(
          jnp.int32, (tnf, W), 0)
      pre = jnp.where(row < n_true, pre, 0.0)
    s = jnp.sum(pre, axis=0, keepdims=True)
    ss = jnp.sum(pre * pre, axis=0, keepdims=True)
    new = jnp.concatenate([s, ss], axis=0).reshape(1, 2, W)

    @pl.when(i == 0)
    def _():
      stats_ref[...] = new

    @pl.when(i > 0)
    def _():
      stats_ref[...] = stats_ref[...] + new

  return body


def _make_pass2(r, q, g0w, giw, inv_n):
  def body(slab_ref, stats_ref, coef_ref, pred_ref, psep_ref):
    st = stats_ref[...]
    s = st[0, 0:1, :] + st[1, 0:1, :]
    ss = st[0, 1:2, :] + st[1, 1:2, :]
    mean = s * inv_n
    var = jnp.maximum(ss * inv_n - mean * mean, 0.0)
    a = jax.lax.rsqrt(var + _BN_EPS) * coef_ref[...]
    b = mean * a
    res = slab_ref[...].astype(jnp.float32) * a - b
    pred_ref[...] = jnp.sum(res, axis=1, keepdims=True)
    psep = res[:, 0:r]
    for e in range(q):
      off = g0w + e * giw
      psep = psep + res[:, off:off + r]
    psep_ref[...] = psep

  return body


def _pack_weights(r, q, h, g0w, giw, gw1, gb1, gw2, gb2, ew1, eb1, ew2, eb2,
                  iw1, ib1, iw2, ib2, coef_g, coef_w, coef_e):
  def padh(a, ha):
    if ha == h:
      return a
    return jnp.concatenate([a, jnp.zeros((h - ha, a.shape[1]), a.dtype)], 0)

  def lanes(g_part, i_part, e_part):
    rows = g_part.shape[0]
    pieces = [g_part, e_part]
    if g0w > r + q:
      pieces.append(jnp.zeros((rows, g0w - r - q), g_part.dtype))
    for e in range(q):
      pieces.append(i_part[:, e * r:(e + 1) * r])
      if giw > r:
        pieces.append(jnp.zeros((rows, giw - r), g_part.dtype))
    return jnp.concatenate(pieces, axis=1).astype(jnp.float32)

  h_g, h_e, h_i = gw1.shape[0], ew1.shape[0], iw1.shape[1]
  iw1f = jnp.transpose(iw1, (1, 0, 2)).reshape(h_i, q * r)
  ib1f = jnp.transpose(ib1, (1, 0, 2)).reshape(h_i, q * r)
  iw2f = jnp.transpose(iw2, (1, 0, 2)).reshape(h_i, q * r)
  ib2f = jnp.transpose(ib2, (1, 0, 2)).reshape(1, q * r)
  w1 = lanes(padh(gw1, h_g), padh(iw1f, h_i), padh(ew1, h_e))
  b1 = lanes(padh(gb1, h_g), padh(ib1f, h_i), padh(eb1, h_e))
  w2 = lanes(padh(gw2, h_g), padh(iw2f, h_i), padh(ew2, h_e))
  b2 = lanes(gb2, ib2f, eb2)
  coef = lanes(coef_g, coef_w.reshape(1, q * r), coef_e)
  return w1, b1, w2, b2, coef


def kernel(X, E, gw1, gb1, gw2, gb2, ew1, eb1, ew2, eb2,
           iw1, ib1, iw2, ib2, coef_g, coef_w, coef_e):
  X = jnp.asarray(X, jnp.float32)
  E = jnp.asarray(E, jnp.float32)
  n, r = X.shape
  q = E.shape[1]
  h = max(gw1.shape[0], ew1.shape[0], iw1.shape[1])
  g0w = _rup(r + q, 128)
  giw = _rup(r, 128)
  W = g0w + q * giw

  w1, b1, w2, b2, coef = _pack_weights(
      r, q, h, g0w, giw, gw1, gb1, gw2, gb2, ew1, eb1, ew2, eb2,
      iw1, ib1, iw2, ib2, coef_g, coef_w, coef_e)
  w1 = w1.astype(jnp.bfloat16)
  b1 = b1.astype(jnp.bfloat16)
  w2 = w2.astype(jnp.bfloat16)
  b2 = b2.astype(jnp.bfloat16)

  if n >= 4096:
    tnf = 2048
  else:
    tnf = max(8, _rup((n + 1) // 2, 8))
  n_pad = _rup(n, 2 * tnf)
  Tc = n_pad // (2 * tnf)
  if n_pad != n:
    X = jnp.concatenate([X, jnp.zeros((n_pad - n, r), jnp.float32)], axis=0)
    E = jnp.concatenate([E, jnp.zeros((n_pad - n, q), jnp.float32)], axis=0)

  vmem_limit = 56 * 2**20

  slab, stats = pl.pallas_call(
      _make_pass1(r, q, h, g0w, giw, tnf, Tc, n),
      out_shape=(jax.ShapeDtypeStruct((n_pad, W), jnp.bfloat16),
                 jax.ShapeDtypeStruct((2, 2, W), jnp.float32)),
      grid=(2, Tc),
      in_specs=[
          pl.BlockSpec((tnf, r), lambda c, i: (c * Tc + i, 0)),
          pl.BlockSpec((tnf, q), lambda c, i: (c * Tc + i, 0)),
          pl.BlockSpec((h, W), lambda c, i: (0, 0)),
          pl.BlockSpec((h, W), lambda c, i: (0, 0)),
          pl.BlockSpec((h, W), lambda c, i: (0, 0)),
          pl.BlockSpec((1, W), lambda c, i: (0, 0)),
      ],
      out_specs=(pl.BlockSpec((tnf, W), lambda c, i: (c * Tc + i, 0)),
                 pl.BlockSpec((1, 2, W), lambda c, i: (c, 0, 0))),
      compiler_params=pltpu.CompilerParams(
          dimension_semantics=("parallel", "arbitrary"),
          vmem_limit_bytes=vmem_limit),
  )(X, E, w1, b1, w2, b2)

  pred, psep = pl.pallas_call(
      _make_pass2(r, q, g0w, giw, 1.0 / float(n)),
      out_shape=(jax.ShapeDtypeStruct((n_pad, 1), jnp.float32),
                 jax.ShapeDtypeStruct((n_pad, r), jnp.float32)),
      grid=(2, Tc),
      in_specs=[
          pl.BlockSpec((tnf, W), lambda c, i: (c * Tc + i, 0)),
          pl.BlockSpec((2, 2, W), lambda c, i: (0, 0, 0)),
          pl.BlockSpec((1, W), lambda c, i: (0, 0)),
      ],
      out_specs=(pl.BlockSpec((tnf, 1), lambda c, i: (c * Tc + i, 0)),
                 pl.BlockSpec((tnf, r), lambda c, i: (c * Tc + i, 0))),
      compiler_params=pltpu.CompilerParams(
          dimension_semantics=("parallel", "arbitrary"),
          vmem_limit_bytes=vmem_limit),
  )(slab, stats, coef)

  if n_pad != n:
    pred = pred[:n]
    psep = psep[:n]
  return pred, psep
